# baseline (speedup 1.0000x reference)
import functools

import jax
import jax.numpy as jnp
from jax import lax
from jax.experimental import pallas as pl
from jax.experimental.pallas import tpu as pltpu
from jax.experimental.pallas import tpu_sc as plsc

N = 2048
LDIM_F = 2047.0
ROWS_PER_BLOCK = 8


def _sort_block(x_ref, val_ref, col_ref):
    x = x_ref[...]
    xb = lax.bitcast_convert_type(x, jnp.int32)
    key = xb & jnp.int32(0x7FFFFFFF)
    sbit = lax.shift_right_logical(xb, 31)
    col = lax.broadcasted_iota(jnp.int32, x.shape, 1)
    v = (col << 1) | sbit

    def cmpex(key, v, j, blk):
        bit = (col & j) != 0
        pk = jnp.where(bit, pltpu.roll(key, j, 1), pltpu.roll(key, N - j, 1))
        pv = jnp.where(bit, pltpu.roll(v, j, 1), pltpu.roll(v, N - j, 1))
        less = (pk < key) | ((pk == key) & (pv < v))
        take = less == (bit == blk)
        return jnp.where(take, pk, key), jnp.where(take, pv, v)

    k = 2
    while k <= N:
        blk = (col & k) != 0
        j = k // 2
        while j >= 1:
            key, v = cmpex(key, v, j, blk)
            j //= 2
        k *= 2

    sfac = 1.0 - 2.0 * (v & 1).astype(jnp.float32)
    absx = lax.bitcast_convert_type(key, jnp.float32)
    xval = absx * sfac
    sgn = jnp.where(key == 0, jnp.float32(0.0), sfac)
    rank = col.astype(jnp.float32)
    val_ref[...] = xval + (rank * sgn) / jnp.float32(LDIM_F)
    col_ref[...] = lax.shift_right_logical(v, 1)


def _tc_sort(x):
    m, n = x.shape
    grid = m // ROWS_PER_BLOCK
    spec = pl.BlockSpec((ROWS_PER_BLOCK, n), lambda i: (i, 0))
    return pl.pallas_call(
        _sort_block,
        grid=(grid,),
        in_specs=[spec],
        out_specs=[spec, spec],
        out_shape=[
            jax.ShapeDtypeStruct((m, n), jnp.float32),
            jax.ShapeDtypeStruct((m, n), jnp.int32),
        ],
    )(x)


def _scatter_rows(val_hbm, col_hbm, out_hbm, idx_v, src_v, buf_v):
    nc = 2
    wid = lax.axis_index("s") * nc + lax.axis_index("c")
    rows_total = out_hbm.shape[0]
    rows_per = rows_total // 32

    def row_body(r, carry):
        row = wid * rows_per + r
        pltpu.sync_copy(col_hbm.at[row], idx_v)
        pltpu.sync_copy(val_hbm.at[row], src_v)

        def chunk(t, c):
            iv = idx_v[pl.ds(t * 16, 16)]
            vv = src_v[pl.ds(t * 16, 16)]
            plsc.store_scatter(buf_v, [iv], vv)
            return c

        lax.fori_loop(0, N // 16, chunk, 0, unroll=4)
        pltpu.sync_copy(buf_v, out_hbm.at[row])
        return carry

    lax.fori_loop(0, rows_per, row_body, 0)


def _sc_scatter(val, colv):
    m, n = val.shape
    mesh = plsc.VectorSubcoreMesh(core_axis_name="c", subcore_axis_name="s")
    return pl.kernel(
        _scatter_rows,
        out_type=jax.ShapeDtypeStruct((m, n), jnp.float32),
        mesh=mesh,
        compiler_params=pltpu.CompilerParams(needs_layout_passes=False),
        scratch_types=[
            pltpu.VMEM((n,), jnp.int32),
            pltpu.VMEM((n,), jnp.float32),
            pltpu.VMEM((n,), jnp.float32),
        ],
    )(val, colv)


def kernel(x):
    val, colv = _tc_sort(x)
    return _sc_scatter(val, colv)


# speedup vs baseline: 1.3225x; 1.3225x over previous
import functools

import jax
import jax.numpy as jnp
from jax import lax
from jax.experimental import pallas as pl
from jax.experimental.pallas import tpu as pltpu
from jax.experimental.pallas import tpu_sc as plsc

N = 2048
LDIM_F = 2047.0
ROWS_PER_BLOCK = 32


def _sort_block(x_ref, val_ref, col_ref):
    x = x_ref[...]
    xb = lax.bitcast_convert_type(x, jnp.int32)
    key = xb & jnp.int32(0x7FFFFFFF)
    sbit = lax.shift_right_logical(xb, 31)
    col = lax.broadcasted_iota(jnp.int32, x.shape, 1)
    v = (col << 1) | sbit

    def cmpex(key, v, j, blk):
        bit = (col & j) != 0
        pk = jnp.where(bit, pltpu.roll(key, j, 1), pltpu.roll(key, N - j, 1))
        pv = jnp.where(bit, pltpu.roll(v, j, 1), pltpu.roll(v, N - j, 1))
        less = (pk < key) | ((pk == key) & (pv < v))
        take = less == (bit == blk)
        return jnp.where(take, pk, key), jnp.where(take, pv, v)

    k = 2
    while k <= N:
        blk = (col & k) != 0
        j = k // 2
        while j >= 1:
            key, v = cmpex(key, v, j, blk)
            j //= 2
        k *= 2

    sfac = 1.0 - 2.0 * (v & 1).astype(jnp.float32)
    absx = lax.bitcast_convert_type(key, jnp.float32)
    xval = absx * sfac
    sgn = jnp.where(key == 0, jnp.float32(0.0), sfac)
    rank = col.astype(jnp.float32)
    val_ref[...] = xval + (rank * sgn) / jnp.float32(LDIM_F)
    col_ref[...] = lax.shift_right_logical(v, 1)


def _tc_sort(x):
    m, n = x.shape
    grid = m // ROWS_PER_BLOCK
    spec = pl.BlockSpec((ROWS_PER_BLOCK, n), lambda i: (i, 0))
    return pl.pallas_call(
        _sort_block,
        grid=(grid,),
        in_specs=[spec],
        out_specs=[spec, spec],
        out_shape=[
            jax.ShapeDtypeStruct((m, n), jnp.float32),
            jax.ShapeDtypeStruct((m, n), jnp.int32),
        ],
    )(x)


def _scatter_rows(val_hbm, col_hbm, out_hbm, idx_v, src_v, buf_v):
    nc = 2
    wid = lax.axis_index("s") * nc + lax.axis_index("c")
    rows_total = out_hbm.shape[0]
    rows_per = rows_total // 32

    def row_body(r, carry):
        row = wid * rows_per + r
        pltpu.sync_copy(col_hbm.at[row], idx_v)
        pltpu.sync_copy(val_hbm.at[row], src_v)

        def chunk(t, c):
            iv = idx_v[pl.ds(t * 16, 16)]
            vv = src_v[pl.ds(t * 16, 16)]
            plsc.store_scatter(buf_v, [iv], vv)
            return c

        lax.fori_loop(0, N // 16, chunk, 0, unroll=4)
        pltpu.sync_copy(buf_v, out_hbm.at[row])
        return carry

    lax.fori_loop(0, rows_per, row_body, 0)


def _sc_scatter(val, colv):
    m, n = val.shape
    mesh = plsc.VectorSubcoreMesh(core_axis_name="c", subcore_axis_name="s")
    return pl.kernel(
        _scatter_rows,
        out_type=jax.ShapeDtypeStruct((m, n), jnp.float32),
        mesh=mesh,
        compiler_params=pltpu.CompilerParams(needs_layout_passes=False),
        scratch_types=[
            pltpu.VMEM((n,), jnp.int32),
            pltpu.VMEM((n,), jnp.float32),
            pltpu.VMEM((n,), jnp.float32),
        ],
    )(val, colv)


def kernel(x):
    val, colv = _tc_sort(x)
    return _sc_scatter(val, colv)
